# Initial kernel scaffold; baseline (speedup 1.0000x reference)
import jax
import jax.numpy as jnp
from jax.experimental import pallas as pl


def kernel(s, W_pi1, b_pi1, g_pi, o_pi, W_pi2, b_pi2, W_v1, b_v1, g_v, o_v, W_v2, b_v2):
    raise NotImplementedError("write your pallas kernel here")


# speedup vs baseline: 13.3708x; 13.3708x over previous
import functools

import jax
import jax.numpy as jnp
from jax import lax
from jax.experimental import pallas as pl
from jax.experimental.pallas import tpu as pltpu
from jax.experimental.pallas import tpu_sc as plsc

B = 4
N = 10000
EMB = 256
FS = 601
NE = 8 * (N - 1)
SLEFT = N * EMB
EPS = 1e-5

NC = 2
NS = 16
CH = 128
NCHUNK = 40
EP = NS * NCHUNK * CH
NROWS = 10240
RPT = NROWS // NS
NT_D = 4
RT = NROWS // NT_D
HALF = EMB // 2


def _fill(ref, n, val, dtype):
    for j in range(n // 16):
        ref[pl.ds(j * 16, 16)] = jnp.full((16,), val, dtype)


def _sc_deg_body(rcv_hbm, deg_hbm, idx_v, ones_v, zer_v, deg_sh):
    c = lax.axis_index("c")
    w = lax.axis_index("s")
    _fill(ones_v, CH, 1.0, jnp.float32)
    _fill(zer_v, RPT, 0.0, jnp.float32)

    def batch(i, car):
        b = 2 * c + i
        pltpu.sync_copy(zer_v, deg_sh.at[pl.ds(w * RPT, RPT)])
        plsc.subcore_barrier()

        def chunk(k, car2):
            eo = (w * NCHUNK + k) * CH
            pltpu.sync_copy(rcv_hbm.at[b, pl.ds(eo, CH)], idx_v)
            pltpu.sync_copy(ones_v, deg_sh.at[idx_v], add=True)
            return car2

        lax.fori_loop(0, NCHUNK, chunk, 0)
        plsc.subcore_barrier()
        pltpu.sync_copy(deg_sh.at[pl.ds(w * RPT, RPT)],
                        deg_hbm.at[b, pl.ds(w * RPT, RPT)])
        plsc.subcore_barrier()
        return car

    lax.fori_loop(0, 2, batch, 0)


_sc_deg = functools.partial(
    pl.kernel,
    mesh=plsc.VectorSubcoreMesh(core_axis_name="c", subcore_axis_name="s"),
    compiler_params=pltpu.CompilerParams(use_tc_tiling_on_sc=False,
                                         needs_layout_passes=False),
    out_type=jax.ShapeDtypeStruct((B, NROWS), jnp.float32),
    scratch_types=[
        pltpu.VMEM((CH,), jnp.int32),
        pltpu.VMEM((CH,), jnp.float32),
        pltpu.VMEM((RPT,), jnp.float32),
        pltpu.VMEM_SHARED((NROWS,), jnp.float32),
    ],
)(_sc_deg_body)


def _dis_body(deg_ref, dis_ref):
    d = deg_ref[...]
    col = lax.broadcasted_iota(jnp.int32, (B, NROWS), 1)
    ok = jnp.logical_and(d > 0, col < N)
    dis_ref[...] = jnp.where(ok, lax.rsqrt(jnp.maximum(d, 1.0)), 0.0)


def _dis_call(deg):
    return pl.pallas_call(
        _dis_body,
        out_shape=jax.ShapeDtypeStruct((B, NROWS), jnp.float32),
    )(deg)


def _prep_body(nodes_ref, dis_ref, out_ref):
    out_ref[0] = nodes_ref[0] * dis_ref[0, 0][:, None]


def _prep_call(nodes, dis3):
    nt = 10
    rows = N // nt
    return pl.pallas_call(
        _prep_body,
        grid=(NC, B, nt),
        in_specs=[
            pl.BlockSpec((1, rows, HALF), lambda c, b, t: (b, t, c)),
            pl.BlockSpec((1, 1, rows), lambda c, b, t: (b * nt + t, 0, 0)),
        ],
        out_specs=pl.BlockSpec((1, rows, HALF), lambda c, b, t: (c, b * nt + t, 0)),
        out_shape=jax.ShapeDtypeStruct((NC, B * N, HALF), jnp.float32),
    )(nodes, dis3)


def _sc_spmm_body(snd_hbm, rcv_hbm, ns_hbm, dis_hbm,
                  p_hbm, sdr_hbm, sds_hbm,
                  sidx, ridx, gidx, rows_v, zrow, zer_v, disv, vsb, vrb,
                  acc_sh, sdr_sh, sds_sh, sem):
    c = lax.axis_index("c")
    w = lax.axis_index("s")

    def zr(i, car):
        for j in range(8):
            zrow[i, pl.ds(j * 16, 16)] = jnp.zeros((16,), jnp.float32)
        return car

    lax.fori_loop(0, CH, zr, 0)
    _fill(zer_v, RPT, 0.0, jnp.float32)

    def batch(b, car):
        def z5(j, car2):
            pltpu.sync_copy(zrow, acc_sh.at[pl.ds(w * RPT + j * CH, CH)])
            return car2

        lax.fori_loop(0, RPT // CH, z5, 0)

        @pl.when(c == 0)
        def _():
            pltpu.sync_copy(zer_v, sdr_sh.at[pl.ds(w * RPT, RPT)])
            pltpu.sync_copy(zer_v, sds_sh.at[pl.ds(w * RPT, RPT)])

        pltpu.sync_copy(dis_hbm.at[b], disv)
        plsc.subcore_barrier()

        base = b * N + c * (B * N)

        def chunk(k, car2):
            eo = (w * NCHUNK + k) * CH
            pltpu.sync_copy(snd_hbm.at[b, pl.ds(eo, CH)], sidx)
            pltpu.sync_copy(rcv_hbm.at[b, pl.ds(eo, CH)], ridx)
            for j in range(CH // 16):
                sl = pl.ds(j * 16, 16)
                gidx[sl] = sidx[sl] + base

            @pl.when(c == 0)
            def _():
                for j in range(CH // 16):
                    sl = pl.ds(j * 16, 16)
                    vsb[sl] = plsc.load_gather(disv, [sidx[sl]])
                    vrb[sl] = plsc.load_gather(disv, [ridx[sl]])
                pltpu.sync_copy(vsb, sdr_sh.at[ridx], add=True)
                pltpu.sync_copy(vrb, sds_sh.at[sidx], add=True)

            pltpu.async_copy(ns_hbm.at[gidx], rows_v, sem).wait()
            pltpu.sync_copy(rows_v, acc_sh.at[ridx], add=True)
            return car2

        lax.fori_loop(0, NCHUNK, chunk, 0)
        plsc.subcore_barrier()
        sl = pl.ds(w * RPT, RPT)
        pltpu.sync_copy(acc_sh.at[sl], p_hbm.at[c, b, sl])

        @pl.when(c == 0)
        def _():
            pltpu.sync_copy(sdr_sh.at[sl], sdr_hbm.at[b, sl])
            pltpu.sync_copy(sds_sh.at[sl], sds_hbm.at[b, sl])

        plsc.subcore_barrier()
        return car

    lax.fori_loop(0, B, batch, 0)


_sc_spmm = functools.partial(
    pl.kernel,
    mesh=plsc.VectorSubcoreMesh(core_axis_name="c", subcore_axis_name="s"),
    compiler_params=pltpu.CompilerParams(use_tc_tiling_on_sc=False,
                                         needs_layout_passes=False),
    out_type=[
        jax.ShapeDtypeStruct((NC, B, NROWS, HALF), jnp.float32),
        jax.ShapeDtypeStruct((B, NROWS), jnp.float32),
        jax.ShapeDtypeStruct((B, NROWS), jnp.float32),
    ],
    scratch_types=[
        pltpu.VMEM((CH,), jnp.int32),
        pltpu.VMEM((CH,), jnp.int32),
        pltpu.VMEM((CH,), jnp.int32),
        pltpu.VMEM((CH, HALF), jnp.float32),
        pltpu.VMEM((CH, HALF), jnp.float32),
        pltpu.VMEM((RPT,), jnp.float32),
        pltpu.VMEM((NROWS,), jnp.float32),
        pltpu.VMEM((CH,), jnp.float32),
        pltpu.VMEM((CH,), jnp.float32),
        pltpu.VMEM_SHARED((NROWS, HALF), jnp.float32),
        pltpu.VMEM_SHARED((NROWS,), jnp.float32),
        pltpu.VMEM_SHARED((NROWS,), jnp.float32),
        pltpu.SemaphoreType.DMA,
    ],
)(_sc_spmm_body)


def _mm_body(p0_ref, p1_ref, dis_ref, sdr_ref, w0_ref, w1_ref, bc_ref,
             z_ref, s1_ref, s2_ref):
    nt = pl.program_id(1)
    z = jnp.dot(p0_ref[0], w0_ref[...], preferred_element_type=jnp.float32)
    z = z + jnp.dot(p1_ref[0], w1_ref[...], preferred_element_type=jnp.float32)
    d = dis_ref[0, 0]
    z = z * d[:, None] + (d * sdr_ref[0, 0])[:, None] * bc_ref[0, 0][None, :]
    z_ref[0] = z
    rows = nt * RT + lax.broadcasted_iota(jnp.int32, (RT, 1), 0)
    zm = jnp.where(rows < N, z, 0.0)
    s1 = jnp.sum(zm, axis=0)
    s2 = jnp.sum(zm * zm, axis=0)

    @pl.when(nt == 0)
    def _():
        s1_ref[0, 0] = s1
        s2_ref[0, 0] = s2

    @pl.when(nt > 0)
    def _():
        s1_ref[0, 0] += s1
        s2_ref[0, 0] += s2


def _mm_call(p0, p1, dis, sdr, w0, w1, bcat):
    nw = w0.shape[1]
    return pl.pallas_call(
        _mm_body,
        grid=(B, NT_D),
        in_specs=[
            pl.BlockSpec((1, RT, HALF), lambda b, t: (b, t, 0)),
            pl.BlockSpec((1, RT, HALF), lambda b, t: (b, t, 0)),
            pl.BlockSpec((1, 1, RT), lambda b, t: (b * NT_D + t, 0, 0)),
            pl.BlockSpec((1, 1, RT), lambda b, t: (b * NT_D + t, 0, 0)),
            pl.BlockSpec((HALF, nw), lambda b, t: (0, 0)),
            pl.BlockSpec((HALF, nw), lambda b, t: (0, 0)),
            pl.BlockSpec((1, 1, nw), lambda b, t: (0, 0, 0)),
        ],
        out_specs=[
            pl.BlockSpec((1, RT, nw), lambda b, t: (b, t, 0)),
            pl.BlockSpec((1, 1, nw), lambda b, t: (b, 0, 0)),
            pl.BlockSpec((1, 1, nw), lambda b, t: (b, 0, 0)),
        ],
        out_shape=[
            jax.ShapeDtypeStruct((B, NROWS, nw), jnp.float32),
            jax.ShapeDtypeStruct((B, 1, nw), jnp.float32),
            jax.ShapeDtypeStruct((B, 1, nw), jnp.float32),
        ],
    )(p0, p1, dis, sdr, w0, w1, bcat)


def _norm_body(z_ref, s1_ref, s2_ref, dis_ref, sds_ref, gc_ref, oc_ref,
               xp_ref, t_ref, sw_ref):
    nt = pl.program_id(1)
    inv = 1.0 / N
    mu = s1_ref[0, 0] * inv
    var = s2_ref[0, 0] * inv - mu * mu
    rstd = lax.rsqrt(var + EPS)
    x = (z_ref[0] - mu[None, :]) * rstd[None, :] * gc_ref[0, 0][None, :] \
        + oc_ref[0, 0][None, :]
    x = jnp.maximum(x, 0.0)
    wv = dis_ref[0, 0] * sds_ref[0, 0]
    tp = jnp.dot(wv[None, :], x, preferred_element_type=jnp.float32)
    xp_ref[0, 0] = x[:, FS]

    @pl.when(nt == 0)
    def _():
        t_ref[0, 0] = tp[0]
        sw_ref[0, 0] = jnp.full(tp[0].shape, jnp.sum(wv), jnp.float32)

    @pl.when(nt > 0)
    def _():
        t_ref[0, 0] += tp[0]
        sw_ref[0, 0] += jnp.full(tp[0].shape, jnp.sum(wv), jnp.float32)


def _norm_call(z, s1, s2, dis, sds, gcat, ocat):
    nw = z.shape[2]
    return pl.pallas_call(
        _norm_body,
        grid=(B, NT_D),
        in_specs=[
            pl.BlockSpec((1, RT, nw), lambda b, t: (b, t, 0)),
            pl.BlockSpec((1, 1, nw), lambda b, t: (b, 0, 0)),
            pl.BlockSpec((1, 1, nw), lambda b, t: (b, 0, 0)),
            pl.BlockSpec((1, 1, RT), lambda b, t: (b * NT_D + t, 0, 0)),
            pl.BlockSpec((1, 1, RT), lambda b, t: (b * NT_D + t, 0, 0)),
            pl.BlockSpec((1, 1, nw), lambda b, t: (0, 0, 0)),
            pl.BlockSpec((1, 1, nw), lambda b, t: (0, 0, 0)),
        ],
        out_specs=[
            pl.BlockSpec((1, 1, RT), lambda b, t: (b * NT_D + t, 0, 0)),
            pl.BlockSpec((1, 1, nw), lambda b, t: (b, 0, 0)),
            pl.BlockSpec((1, 1, nw), lambda b, t: (b, 0, 0)),
        ],
        out_shape=[
            jax.ShapeDtypeStruct((B * NT_D, 1, RT), jnp.float32),
            jax.ShapeDtypeStruct((B, 1, nw), jnp.float32),
            jax.ShapeDtypeStruct((B, 1, nw), jnp.float32),
        ],
    )(z, s1, s2, dis, sds, gcat, ocat)


def _sc_pol_body(snd_hbm, rcv_hbm, dis_hbm, xp_hbm, sdr_hbm, wb_hbm,
                 lg_hbm,
                 sidx, ridx, msg, disv, xpv, accv, sdrv, outv, zer_v, wbv,
                 acc_sh):
    c = lax.axis_index("c")
    w = lax.axis_index("s")
    _fill(zer_v, RPT, 0.0, jnp.float32)
    pltpu.sync_copy(wb_hbm, wbv)

    def batch(i, car):
        b = 2 * c + i
        pltpu.sync_copy(zer_v, acc_sh.at[pl.ds(w * RPT, RPT)])
        pltpu.sync_copy(dis_hbm.at[b], disv)
        pltpu.sync_copy(xp_hbm.at[b], xpv)
        pltpu.sync_copy(sdr_hbm.at[b, pl.ds(w * RPT, RPT)], sdrv)
        plsc.subcore_barrier()

        def chunk(k, car2):
            eo = (w * NCHUNK + k) * CH
            pltpu.sync_copy(snd_hbm.at[b, pl.ds(eo, CH)], sidx)
            pltpu.sync_copy(rcv_hbm.at[b, pl.ds(eo, CH)], ridx)
            for j in range(CH // 16):
                sl = pl.ds(j * 16, 16)
                sv = sidx[sl]
                msg[sl] = (plsc.load_gather(disv, [sv])
                           * plsc.load_gather(disv, [ridx[sl]])
                           * plsc.load_gather(xpv, [sv]))
            pltpu.sync_copy(msg, acc_sh.at[ridx], add=True)
            return car2

        lax.fori_loop(0, NCHUNK, chunk, 0)
        plsc.subcore_barrier()
        pltpu.sync_copy(acc_sh.at[pl.ds(w * RPT, RPT)], accv)
        wvec = wbv[...]
        w2 = wvec[0]
        b2 = wvec[1]

        def vv(j, car2):
            sl = pl.ds(j * 16, 16)
            dsl = disv[pl.ds(w * RPT + j * 16, 16)]
            outv[sl] = accv[sl] * w2 + dsl * sdrv[sl] * b2
            return car2

        lax.fori_loop(0, RPT // 16, vv, 0)
        pltpu.sync_copy(outv, lg_hbm.at[b, pl.ds(w * RPT, RPT)])
        plsc.subcore_barrier()
        return car

    lax.fori_loop(0, 2, batch, 0)


_sc_pol = functools.partial(
    pl.kernel,
    mesh=plsc.VectorSubcoreMesh(core_axis_name="c", subcore_axis_name="s"),
    compiler_params=pltpu.CompilerParams(use_tc_tiling_on_sc=False,
                                         needs_layout_passes=False),
    out_type=jax.ShapeDtypeStruct((B, NROWS), jnp.float32),
    scratch_types=[
        pltpu.VMEM((CH,), jnp.int32),
        pltpu.VMEM((CH,), jnp.int32),
        pltpu.VMEM((CH,), jnp.float32),
        pltpu.VMEM((NROWS,), jnp.float32),
        pltpu.VMEM((NROWS,), jnp.float32),
        pltpu.VMEM((RPT,), jnp.float32),
        pltpu.VMEM((RPT,), jnp.float32),
        pltpu.VMEM((RPT,), jnp.float32),
        pltpu.VMEM((RPT,), jnp.float32),
        pltpu.VMEM((16,), jnp.float32),
        pltpu.VMEM_SHARED((NROWS,), jnp.float32),
    ],
)(_sc_pol_body)


def _fin_body(t_ref, w2_ref, sw_ref, bv_ref, v_ref):
    v_ref[...] = jnp.dot(t_ref[...], w2_ref[...],
                         preferred_element_type=jnp.float32) \
        + sw_ref[:, 0:1] * bv_ref[...]


def _fin_call(t, w2p, sw, bv2):
    return pl.pallas_call(
        _fin_body,
        out_shape=jax.ShapeDtypeStruct((B, FS), jnp.float32),
    )(t, w2p, sw, bv2)


def kernel(s, W_pi1, b_pi1, g_pi, o_pi, W_pi2, b_pi2, W_v1, b_v1, g_v, o_v,
           W_v2, b_v2):
    f32 = jnp.float32
    nodes = s[:, :SLEFT].reshape(B, N, EMB)
    snd = s[:, SLEFT:SLEFT + NE].astype(jnp.int32)
    rcv = s[:, SLEFT + NE:SLEFT + 2 * NE].astype(jnp.int32)
    pad = EP - NE
    snd_p = jnp.pad(snd, ((0, 0), (0, pad)))
    rcv_pad = (N + (jnp.arange(pad, dtype=jnp.int32) % 16))[None, :]
    rcv_p = jnp.concatenate([rcv, jnp.broadcast_to(rcv_pad, (B, pad))], axis=1)

    deg = _sc_deg(rcv_p)
    dis = _dis_call(deg)
    ns = _prep_call(nodes, dis[:, :N].reshape(B * 10, 1, N // 10)).reshape(
        NC * B * N, HALF)
    p, sdr, sds = _sc_spmm(snd_p, rcv_p, ns, dis)

    nw = 640
    wcat = jnp.concatenate([W_v1, W_pi1], axis=1)
    wcat = jnp.pad(wcat, ((0, 0), (0, nw - FS - 1)))
    bcat = jnp.pad(jnp.concatenate([b_v1, b_pi1]), (0, nw - FS - 1))[None, None, :]
    gcat = jnp.pad(jnp.concatenate([g_v, g_pi]), (0, nw - FS - 1))[None, None, :]
    ocat = jnp.pad(jnp.concatenate([o_v, o_pi]), (0, nw - FS - 1))[None, None, :]

    dis_rt = dis.reshape(B * NT_D, 1, RT)
    sdr_rt = sdr.reshape(B * NT_D, 1, RT)
    sds_rt = sds.reshape(B * NT_D, 1, RT)
    z, s1, s2 = _mm_call(p[0], p[1], dis_rt, sdr_rt,
                         wcat[:HALF], wcat[HALF:], bcat)
    xp3, t3, sw3 = _norm_call(z, s1, s2, dis_rt, sds_rt, gcat, ocat)
    xp = xp3.reshape(B, NROWS)
    t = t3.reshape(B, nw)
    sw = sw3.reshape(B, nw)

    wb = jnp.zeros((16,), f32).at[0].set(W_pi2[0, 0]).at[1].set(b_pi2[0])
    lg = _sc_pol(snd_p, rcv_p, dis, xp, sdr, wb)
    logits = lg[:, :N - 2]

    w2p = jnp.pad(W_v2, ((0, nw - FS), (0, 0)))
    v = _fin_call(t, w2p, sw, b_v2[None, :])
    return (v, logits)


# speedup vs baseline: 17.4900x; 1.3081x over previous
import functools

import jax
import jax.numpy as jnp
from jax import lax
from jax.experimental import pallas as pl
from jax.experimental.pallas import tpu as pltpu
from jax.experimental.pallas import tpu_sc as plsc

B = 4
N = 10000
EMB = 256
FS = 601
NE = 8 * (N - 1)
SLEFT = N * EMB
EPS = 1e-5

NC = 2
NS = 16
CH = 128
NCHUNK = 40
EP = NS * NCHUNK * CH
NROWS = 10240
RPT = NROWS // NS
NT_D = 4
RT = NROWS // NT_D
HALF = EMB // 2


def _fill(ref, n, val, dtype):
    for j in range(n // 16):
        ref[pl.ds(j * 16, 16)] = jnp.full((16,), val, dtype)


def _rsqrt16(x):
    i = plsc.bitcast(x, jnp.int32)
    i = jnp.full((16,), 0x5F3759DF, jnp.int32) - lax.shift_right_arithmetic(i, 1)
    y = plsc.bitcast(i, jnp.float32)
    for _ in range(3):
        y = y * (1.5 - 0.5 * x * y * y)
    return y


def _sc_pre_body(snd2_hbm, rcv2_hbm, dis_hbm, sdr_hbm, sds_hbm,
                 si2, ri2, ones_v, zer_v, degv, disv, vb0, vb1, vb2, vb3,
                 deg_sh, sdr_sh, sds_sh, semd, sem0, sem1):
    c = lax.axis_index("c")
    w = lax.axis_index("s")
    _fill(ones_v, CH, 1.0, jnp.float32)
    _fill(zer_v, RPT, 0.0, jnp.float32)

    def batch(i, car):
        b = 2 * c + i
        sl = pl.ds(w * RPT, RPT)
        pltpu.sync_copy(zer_v, deg_sh.at[sl])
        pltpu.sync_copy(zer_v, sdr_sh.at[sl])
        pltpu.sync_copy(zer_v, sds_sh.at[sl])
        pltpu.sync_copy(snd2_hbm.at[b, pl.ds(w * NCHUNK, NCHUNK)], si2)
        pltpu.sync_copy(rcv2_hbm.at[b, pl.ds(w * NCHUNK, NCHUNK)], ri2)
        plsc.subcore_barrier()
        descs = [pltpu.async_copy(ones_v, deg_sh.at[ri2.at[k]], semd,
                                  add=True) for k in range(NCHUNK)]
        for d in descs:
            d.wait()
        plsc.subcore_barrier()
        pltpu.sync_copy(deg_sh, degv)

        def dv(j, car2):
            s16 = pl.ds(j * 16, 16)
            d = degv[s16]
            row = jnp.full((16,), 16, jnp.int32) * j + lax.iota(jnp.int32, 16)
            y = _rsqrt16(jnp.maximum(d, 1.0))
            ok = jnp.logical_and(d > 0.0, row < N)
            disv[s16] = jnp.where(ok, y, 0.0)
            return car2

        lax.fori_loop(0, NROWS // 16, dv, 0)
        pltpu.sync_copy(disv.at[sl], dis_hbm.at[b, sl])
        dprev = [None, None]
        for k in range(NCHUNK):
            p = k & 1
            vs = vb0 if p == 0 else vb1
            vr = vb2 if p == 0 else vb3
            if dprev[p] is not None:
                dprev[p][0].wait()
                dprev[p][1].wait()
            for j in range(CH // 16):
                s16 = pl.ds(j * 16, 16)
                vs[s16] = plsc.load_gather(disv, [si2[k, s16]])
                vr[s16] = plsc.load_gather(disv, [ri2[k, s16]])
            sem = sem0 if p == 0 else sem1
            d1 = pltpu.async_copy(vs, sdr_sh.at[ri2.at[k]], sem, add=True)
            d2 = pltpu.async_copy(vr, sds_sh.at[si2.at[k]], sem, add=True)
            dprev[p] = (d1, d2)
        for p in range(2):
            dprev[p][0].wait()
            dprev[p][1].wait()
        plsc.subcore_barrier()
        pltpu.sync_copy(sdr_sh.at[sl], sdr_hbm.at[b, sl])
        pltpu.sync_copy(sds_sh.at[sl], sds_hbm.at[b, sl])
        plsc.subcore_barrier()
        return car

    lax.fori_loop(0, 2, batch, 0)


_sc_pre = functools.partial(
    pl.kernel,
    mesh=plsc.VectorSubcoreMesh(core_axis_name="c", subcore_axis_name="s"),
    compiler_params=pltpu.CompilerParams(use_tc_tiling_on_sc=False,
                                         needs_layout_passes=False),
    out_type=[
        jax.ShapeDtypeStruct((B, NROWS), jnp.float32),
        jax.ShapeDtypeStruct((B, NROWS), jnp.float32),
        jax.ShapeDtypeStruct((B, NROWS), jnp.float32),
    ],
    scratch_types=[
        pltpu.VMEM((NCHUNK, CH), jnp.int32),
        pltpu.VMEM((NCHUNK, CH), jnp.int32),
        pltpu.VMEM((CH,), jnp.float32),
        pltpu.VMEM((RPT,), jnp.float32),
        pltpu.VMEM((NROWS,), jnp.float32),
        pltpu.VMEM((NROWS,), jnp.float32),
        pltpu.VMEM((CH,), jnp.float32),
        pltpu.VMEM((CH,), jnp.float32),
        pltpu.VMEM((CH,), jnp.float32),
        pltpu.VMEM((CH,), jnp.float32),
        pltpu.VMEM_SHARED((NROWS,), jnp.float32),
        pltpu.VMEM_SHARED((NROWS,), jnp.float32),
        pltpu.VMEM_SHARED((NROWS,), jnp.float32),
        pltpu.SemaphoreType.DMA,
        pltpu.SemaphoreType.DMA,
        pltpu.SemaphoreType.DMA,
    ],
)(_sc_pre_body)


def _prep_body(nodes_ref, dis_ref, out_ref):
    out_ref[0] = nodes_ref[0] * dis_ref[0, 0][:, None]


def _prep_call(nodes, dis3):
    nt = 10
    rows = N // nt
    return pl.pallas_call(
        _prep_body,
        grid=(NC, B, nt),
        in_specs=[
            pl.BlockSpec((1, rows, HALF), lambda c, b, t: (b, t, c)),
            pl.BlockSpec((1, 1, rows), lambda c, b, t: (b * nt + t, 0, 0)),
        ],
        out_specs=pl.BlockSpec((1, rows, HALF), lambda c, b, t: (c, b * nt + t, 0)),
        out_shape=jax.ShapeDtypeStruct((NC, B * N, HALF), jnp.float32),
    )(nodes, dis3)


EPT = NCHUNK * CH


def _sc_spmm_body(snd2_hbm, rcv2_hbm, ns_hbm, z_hbm,
                  p_hbm,
                  si2, ri2, rows0, rows1,
                  acc_sh, g0, g1, s0, s1):
    c = lax.axis_index("c")
    w = lax.axis_index("s")

    def batch(b, car):
        for j in range(RPT // CH):
            pltpu.sync_copy(z_hbm, acc_sh.at[pl.ds(w * RPT + j * CH, CH)])
        pltpu.sync_copy(snd2_hbm.at[b, pl.ds(w * NCHUNK, NCHUNK)], si2)
        pltpu.sync_copy(rcv2_hbm.at[b, pl.ds(w * NCHUNK, NCHUNK)], ri2)
        base = b * N + c * (B * N)

        def gi(k, car2):
            for j in range(CH // 16):
                sl = pl.ds(j * 16, 16)
                si2[k, sl] = si2[k, sl] + base
            return car2

        lax.fori_loop(0, NCHUNK, gi, 0)
        plsc.subcore_barrier()

        bufs = (rows0, rows1)
        gsem = (g0, g1)
        ssem = (s0, s1)
        gd = [pltpu.async_copy(ns_hbm.at[si2.at[k]], bufs[k], gsem[k])
              for k in range(2)]
        sd = [None, None]
        for k in range(NCHUNK):
            p = k & 1
            gd[p].wait()
            sd[p] = pltpu.async_copy(bufs[p], acc_sh.at[ri2.at[k]],
                                     ssem[p], add=True)
            if k + 2 < NCHUNK:
                sd[p].wait()
                gd[p] = pltpu.async_copy(ns_hbm.at[si2.at[k + 2]],
                                         bufs[p], gsem[p])
        sd[0].wait()
        sd[1].wait()
        plsc.subcore_barrier()
        sl = pl.ds(w * RPT, RPT)
        pltpu.sync_copy(acc_sh.at[sl], p_hbm.at[c, b, sl])
        plsc.subcore_barrier()
        return car

    lax.fori_loop(0, B, batch, 0)


_sc_spmm = functools.partial(
    pl.kernel,
    mesh=plsc.VectorSubcoreMesh(core_axis_name="c", subcore_axis_name="s"),
    compiler_params=pltpu.CompilerParams(use_tc_tiling_on_sc=False,
                                         needs_layout_passes=False),
    out_type=jax.ShapeDtypeStruct((NC, B, NROWS, HALF), jnp.float32),
    scratch_types=[
        pltpu.VMEM((NCHUNK, CH), jnp.int32),
        pltpu.VMEM((NCHUNK, CH), jnp.int32),
        pltpu.VMEM((CH, HALF), jnp.float32),
        pltpu.VMEM((CH, HALF), jnp.float32),
        pltpu.VMEM_SHARED((NROWS, HALF), jnp.float32),
        pltpu.SemaphoreType.DMA,
        pltpu.SemaphoreType.DMA,
        pltpu.SemaphoreType.DMA,
        pltpu.SemaphoreType.DMA,
    ],
)(_sc_spmm_body)


def _mm_body(p0_ref, p1_ref, dis_ref, sdr_ref, w0_ref, w1_ref, bc_ref,
             z_ref, s1_ref, s2_ref):
    nt = pl.program_id(1)
    z = jnp.dot(p0_ref[0], w0_ref[...], preferred_element_type=jnp.float32)
    z = z + jnp.dot(p1_ref[0], w1_ref[...], preferred_element_type=jnp.float32)
    d = dis_ref[0, 0]
    z = z * d[:, None] + (d * sdr_ref[0, 0])[:, None] * bc_ref[0, 0][None, :]
    z_ref[0] = z
    rows = nt * RT + lax.broadcasted_iota(jnp.int32, (RT, 1), 0)
    zm = jnp.where(rows < N, z, 0.0)
    s1 = jnp.sum(zm, axis=0)
    s2 = jnp.sum(zm * zm, axis=0)

    @pl.when(nt == 0)
    def _():
        s1_ref[0, 0] = s1
        s2_ref[0, 0] = s2

    @pl.when(nt > 0)
    def _():
        s1_ref[0, 0] += s1
        s2_ref[0, 0] += s2


def _mm_call(p0, p1, dis, sdr, w0, w1, bcat):
    nw = w0.shape[1]
    return pl.pallas_call(
        _mm_body,
        grid=(B, NT_D),
        in_specs=[
            pl.BlockSpec((1, RT, HALF), lambda b, t: (b, t, 0)),
            pl.BlockSpec((1, RT, HALF), lambda b, t: (b, t, 0)),
            pl.BlockSpec((1, 1, RT), lambda b, t: (b * NT_D + t, 0, 0)),
            pl.BlockSpec((1, 1, RT), lambda b, t: (b * NT_D + t, 0, 0)),
            pl.BlockSpec((HALF, nw), lambda b, t: (0, 0)),
            pl.BlockSpec((HALF, nw), lambda b, t: (0, 0)),
            pl.BlockSpec((1, 1, nw), lambda b, t: (0, 0, 0)),
        ],
        out_specs=[
            pl.BlockSpec((1, RT, nw), lambda b, t: (b, t, 0)),
            pl.BlockSpec((1, 1, nw), lambda b, t: (b, 0, 0)),
            pl.BlockSpec((1, 1, nw), lambda b, t: (b, 0, 0)),
        ],
        out_shape=[
            jax.ShapeDtypeStruct((B, NROWS, nw), jnp.float32),
            jax.ShapeDtypeStruct((B, 1, nw), jnp.float32),
            jax.ShapeDtypeStruct((B, 1, nw), jnp.float32),
        ],
    )(p0, p1, dis, sdr, w0, w1, bcat)


def _norm_body(z_ref, s1_ref, s2_ref, dis_ref, sds_ref, gc_ref, oc_ref,
               xp_ref, t_ref, sw_ref):
    nt = pl.program_id(1)
    inv = 1.0 / N
    mu = s1_ref[0, 0] * inv
    var = s2_ref[0, 0] * inv - mu * mu
    rstd = lax.rsqrt(var + EPS)
    x = (z_ref[0] - mu[None, :]) * rstd[None, :] * gc_ref[0, 0][None, :] \
        + oc_ref[0, 0][None, :]
    x = jnp.maximum(x, 0.0)
    wv = dis_ref[0, 0] * sds_ref[0, 0]
    tp = jnp.dot(wv[None, :], x, preferred_element_type=jnp.float32)
    xp_ref[0, 0] = x[:, FS]

    @pl.when(nt == 0)
    def _():
        t_ref[0, 0] = tp[0]
        sw_ref[0, 0] = jnp.full(tp[0].shape, jnp.sum(wv), jnp.float32)

    @pl.when(nt > 0)
    def _():
        t_ref[0, 0] += tp[0]
        sw_ref[0, 0] += jnp.full(tp[0].shape, jnp.sum(wv), jnp.float32)


def _norm_call(z, s1, s2, dis, sds, gcat, ocat):
    nw = z.shape[2]
    return pl.pallas_call(
        _norm_body,
        grid=(B, NT_D),
        in_specs=[
            pl.BlockSpec((1, RT, nw), lambda b, t: (b, t, 0)),
            pl.BlockSpec((1, 1, nw), lambda b, t: (b, 0, 0)),
            pl.BlockSpec((1, 1, nw), lambda b, t: (b, 0, 0)),
            pl.BlockSpec((1, 1, RT), lambda b, t: (b * NT_D + t, 0, 0)),
            pl.BlockSpec((1, 1, RT), lambda b, t: (b * NT_D + t, 0, 0)),
            pl.BlockSpec((1, 1, nw), lambda b, t: (0, 0, 0)),
            pl.BlockSpec((1, 1, nw), lambda b, t: (0, 0, 0)),
        ],
        out_specs=[
            pl.BlockSpec((1, 1, RT), lambda b, t: (b * NT_D + t, 0, 0)),
            pl.BlockSpec((1, 1, nw), lambda b, t: (b, 0, 0)),
            pl.BlockSpec((1, 1, nw), lambda b, t: (b, 0, 0)),
        ],
        out_shape=[
            jax.ShapeDtypeStruct((B * NT_D, 1, RT), jnp.float32),
            jax.ShapeDtypeStruct((B, 1, nw), jnp.float32),
            jax.ShapeDtypeStruct((B, 1, nw), jnp.float32),
        ],
    )(z, s1, s2, dis, sds, gcat, ocat)


def _sc_pol_body(snd_hbm, rcv_hbm, dis_hbm, xp_hbm, sdr_hbm, wb_hbm,
                 lg_hbm,
                 sidx, ridx, msg, disv, xpv, accv, sdrv, outv, zer_v, wbv,
                 acc_sh):
    c = lax.axis_index("c")
    w = lax.axis_index("s")
    _fill(zer_v, RPT, 0.0, jnp.float32)
    pltpu.sync_copy(wb_hbm, wbv)

    def batch(i, car):
        b = 2 * c + i
        pltpu.sync_copy(zer_v, acc_sh.at[pl.ds(w * RPT, RPT)])
        pltpu.sync_copy(dis_hbm.at[b], disv)
        pltpu.sync_copy(xp_hbm.at[b], xpv)
        pltpu.sync_copy(sdr_hbm.at[b, pl.ds(w * RPT, RPT)], sdrv)
        plsc.subcore_barrier()

        def chunk(k, car2):
            eo = (w * NCHUNK + k) * CH
            pltpu.sync_copy(snd_hbm.at[b, pl.ds(eo, CH)], sidx)
            pltpu.sync_copy(rcv_hbm.at[b, pl.ds(eo, CH)], ridx)
            for j in range(CH // 16):
                sl = pl.ds(j * 16, 16)
                sv = sidx[sl]
                msg[sl] = (plsc.load_gather(disv, [sv])
                           * plsc.load_gather(disv, [ridx[sl]])
                           * plsc.load_gather(xpv, [sv]))
            pltpu.sync_copy(msg, acc_sh.at[ridx], add=True)
            return car2

        lax.fori_loop(0, NCHUNK, chunk, 0)
        plsc.subcore_barrier()
        pltpu.sync_copy(acc_sh.at[pl.ds(w * RPT, RPT)], accv)
        wvec = wbv[...]
        w2 = wvec[0]
        b2 = wvec[1]

        def vv(j, car2):
            sl = pl.ds(j * 16, 16)
            dsl = disv[pl.ds(w * RPT + j * 16, 16)]
            outv[sl] = accv[sl] * w2 + dsl * sdrv[sl] * b2
            return car2

        lax.fori_loop(0, RPT // 16, vv, 0)
        pltpu.sync_copy(outv, lg_hbm.at[b, pl.ds(w * RPT, RPT)])
        plsc.subcore_barrier()
        return car

    lax.fori_loop(0, 2, batch, 0)


_sc_pol = functools.partial(
    pl.kernel,
    mesh=plsc.VectorSubcoreMesh(core_axis_name="c", subcore_axis_name="s"),
    compiler_params=pltpu.CompilerParams(use_tc_tiling_on_sc=False,
                                         needs_layout_passes=False),
    out_type=jax.ShapeDtypeStruct((B, NROWS), jnp.float32),
    scratch_types=[
        pltpu.VMEM((CH,), jnp.int32),
        pltpu.VMEM((CH,), jnp.int32),
        pltpu.VMEM((CH,), jnp.float32),
        pltpu.VMEM((NROWS,), jnp.float32),
        pltpu.VMEM((NROWS,), jnp.float32),
        pltpu.VMEM((RPT,), jnp.float32),
        pltpu.VMEM((RPT,), jnp.float32),
        pltpu.VMEM((RPT,), jnp.float32),
        pltpu.VMEM((RPT,), jnp.float32),
        pltpu.VMEM((16,), jnp.float32),
        pltpu.VMEM_SHARED((NROWS,), jnp.float32),
    ],
)(_sc_pol_body)


def _fin_body(t_ref, w2_ref, sw_ref, bv_ref, v_ref):
    v_ref[...] = jnp.dot(t_ref[...], w2_ref[...],
                         preferred_element_type=jnp.float32) \
        + sw_ref[:, 0:1] * bv_ref[...]


def _fin_call(t, w2p, sw, bv2):
    return pl.pallas_call(
        _fin_body,
        out_shape=jax.ShapeDtypeStruct((B, FS), jnp.float32),
    )(t, w2p, sw, bv2)


def kernel(s, W_pi1, b_pi1, g_pi, o_pi, W_pi2, b_pi2, W_v1, b_v1, g_v, o_v,
           W_v2, b_v2):
    f32 = jnp.float32
    nodes = s[:, :SLEFT].reshape(B, N, EMB)
    snd = s[:, SLEFT:SLEFT + NE].astype(jnp.int32)
    rcv = s[:, SLEFT + NE:SLEFT + 2 * NE].astype(jnp.int32)
    pad = EP - NE
    snd_p = jnp.pad(snd, ((0, 0), (0, pad)))
    rcv_pad = (N + (jnp.arange(pad, dtype=jnp.int32) % 16))[None, :]
    rcv_p = jnp.concatenate([rcv, jnp.broadcast_to(rcv_pad, (B, pad))], axis=1)

    snd3 = snd_p.reshape(B, NS * NCHUNK, CH)
    rcv3 = rcv_p.reshape(B, NS * NCHUNK, CH)
    dis, sdr, sds = _sc_pre(snd3, rcv3)
    ns = _prep_call(nodes, dis[:, :N].reshape(B * 10, 1, N // 10)).reshape(
        NC * B * N, HALF)
    z128 = jnp.zeros((CH, HALF), f32)
    p = _sc_spmm(snd3, rcv3, ns, z128)

    nw = 640
    wcat = jnp.concatenate([W_v1, W_pi1], axis=1)
    wcat = jnp.pad(wcat, ((0, 0), (0, nw - FS - 1)))
    bcat = jnp.pad(jnp.concatenate([b_v1, b_pi1]), (0, nw - FS - 1))[None, None, :]
    gcat = jnp.pad(jnp.concatenate([g_v, g_pi]), (0, nw - FS - 1))[None, None, :]
    ocat = jnp.pad(jnp.concatenate([o_v, o_pi]), (0, nw - FS - 1))[None, None, :]

    dis_rt = dis.reshape(B * NT_D, 1, RT)
    sdr_rt = sdr.reshape(B * NT_D, 1, RT)
    sds_rt = sds.reshape(B * NT_D, 1, RT)
    z, s1, s2 = _mm_call(p[0], p[1], dis_rt, sdr_rt,
                         wcat[:HALF], wcat[HALF:], bcat)
    xp3, t3, sw3 = _norm_call(z, s1, s2, dis_rt, sds_rt, gcat, ocat)
    xp = xp3.reshape(B, NROWS)
    t = t3.reshape(B, nw)
    sw = sw3.reshape(B, nw)

    wb = jnp.zeros((16,), f32).at[0].set(W_pi2[0, 0]).at[1].set(b_pi2[0])
    lg = _sc_pol(snd_p, rcv_p, dis, xp, sdr, wb)
    logits = lg[:, :N - 2]

    w2p = jnp.pad(W_v2, ((0, nw - FS), (0, 0)))
    v = _fin_call(t, w2p, sw, b_v2[None, :])
    return (v, logits)


# speedup vs baseline: 18.8238x; 1.0763x over previous
import functools

import jax
import jax.numpy as jnp
from jax import lax
from jax.experimental import pallas as pl
from jax.experimental.pallas import tpu as pltpu
from jax.experimental.pallas import tpu_sc as plsc

B = 4
N = 10000
EMB = 256
FS = 601
NE = 8 * (N - 1)
SLEFT = N * EMB
EPS = 1e-5

NC = 2
NS = 16
CH = 128
NCHUNK = 40
EP = NS * NCHUNK * CH
NROWS = 10240
RPT = NROWS // NS
NT_D = 4
RT = NROWS // NT_D
HALF = EMB // 2


def _fill(ref, n, val, dtype):
    for j in range(n // 16):
        ref[pl.ds(j * 16, 16)] = jnp.full((16,), val, dtype)


def _rsqrt16(x):
    i = plsc.bitcast(x, jnp.int32)
    i = jnp.full((16,), 0x5F3759DF, jnp.int32) - lax.shift_right_arithmetic(i, 1)
    y = plsc.bitcast(i, jnp.float32)
    for _ in range(3):
        y = y * (1.5 - 0.5 * x * y * y)
    return y


def _sc_pre_body(snd2_hbm, rcv2_hbm, dis_hbm, sdr_hbm, sds_hbm,
                 si2, ri2, ones_v, zer_v, degv, disv, vb0, vb1, vb2, vb3,
                 deg_sh, sdr_sh, sds_sh, semd, sem0, sem1):
    c = lax.axis_index("c")
    w = lax.axis_index("s")
    _fill(ones_v, CH, 1.0, jnp.float32)
    _fill(zer_v, RPT, 0.0, jnp.float32)

    def batch(i, car):
        b = 2 * c + i
        sl = pl.ds(w * RPT, RPT)
        pltpu.sync_copy(zer_v, deg_sh.at[sl])
        pltpu.sync_copy(zer_v, sdr_sh.at[sl])
        pltpu.sync_copy(zer_v, sds_sh.at[sl])
        pltpu.sync_copy(snd2_hbm.at[b, pl.ds(w * NCHUNK, NCHUNK)], si2)
        pltpu.sync_copy(rcv2_hbm.at[b, pl.ds(w * NCHUNK, NCHUNK)], ri2)
        plsc.subcore_barrier()
        descs = [pltpu.async_copy(ones_v, deg_sh.at[ri2.at[k]], semd,
                                  add=True) for k in range(NCHUNK)]
        for d in descs:
            d.wait()
        plsc.subcore_barrier()
        pltpu.sync_copy(deg_sh, degv)

        def dv(j, car2):
            s16 = pl.ds(j * 16, 16)
            d = degv[s16]
            row = jnp.full((16,), 16, jnp.int32) * j + lax.iota(jnp.int32, 16)
            y = _rsqrt16(jnp.maximum(d, 1.0))
            ok = jnp.logical_and(d > 0.0, row < N)
            disv[s16] = jnp.where(ok, y, 0.0)
            return car2

        lax.fori_loop(0, NROWS // 16, dv, 0)
        pltpu.sync_copy(disv.at[sl], dis_hbm.at[b, sl])
        dprev = [None, None]
        for k in range(NCHUNK):
            p = k & 1
            vs = vb0 if p == 0 else vb1
            vr = vb2 if p == 0 else vb3
            if dprev[p] is not None:
                dprev[p][0].wait()
                dprev[p][1].wait()
            for j in range(CH // 16):
                s16 = pl.ds(j * 16, 16)
                vs[s16] = plsc.load_gather(disv, [si2[k, s16]])
                vr[s16] = plsc.load_gather(disv, [ri2[k, s16]])
            sem = sem0 if p == 0 else sem1
            d1 = pltpu.async_copy(vs, sdr_sh.at[ri2.at[k]], sem, add=True)
            d2 = pltpu.async_copy(vr, sds_sh.at[si2.at[k]], sem, add=True)
            dprev[p] = (d1, d2)
        for p in range(2):
            dprev[p][0].wait()
            dprev[p][1].wait()
        plsc.subcore_barrier()
        pltpu.sync_copy(sdr_sh.at[sl], sdr_hbm.at[b, sl])
        pltpu.sync_copy(sds_sh.at[sl], sds_hbm.at[b, sl])
        plsc.subcore_barrier()
        return car

    lax.fori_loop(0, 2, batch, 0)


_sc_pre = functools.partial(
    pl.kernel,
    mesh=plsc.VectorSubcoreMesh(core_axis_name="c", subcore_axis_name="s"),
    compiler_params=pltpu.CompilerParams(use_tc_tiling_on_sc=False,
                                         needs_layout_passes=False),
    out_type=[
        jax.ShapeDtypeStruct((B, NROWS), jnp.float32),
        jax.ShapeDtypeStruct((B, NROWS), jnp.float32),
        jax.ShapeDtypeStruct((B, NROWS), jnp.float32),
    ],
    scratch_types=[
        pltpu.VMEM((NCHUNK, CH), jnp.int32),
        pltpu.VMEM((NCHUNK, CH), jnp.int32),
        pltpu.VMEM((CH,), jnp.float32),
        pltpu.VMEM((RPT,), jnp.float32),
        pltpu.VMEM((NROWS,), jnp.float32),
        pltpu.VMEM((NROWS,), jnp.float32),
        pltpu.VMEM((CH,), jnp.float32),
        pltpu.VMEM((CH,), jnp.float32),
        pltpu.VMEM((CH,), jnp.float32),
        pltpu.VMEM((CH,), jnp.float32),
        pltpu.VMEM_SHARED((NROWS,), jnp.float32),
        pltpu.VMEM_SHARED((NROWS,), jnp.float32),
        pltpu.VMEM_SHARED((NROWS,), jnp.float32),
        pltpu.SemaphoreType.DMA,
        pltpu.SemaphoreType.DMA,
        pltpu.SemaphoreType.DMA,
    ],
)(_sc_pre_body)


def _prep_body(nodes_ref, dis_ref, out_ref):
    out_ref[0] = nodes_ref[0] * dis_ref[0, 0][:, None]


def _prep_call(nodes, dis3):
    nt = 10
    rows = N // nt
    return pl.pallas_call(
        _prep_body,
        grid=(NC, B, nt),
        in_specs=[
            pl.BlockSpec((1, rows, HALF), lambda c, b, t: (b, t, c)),
            pl.BlockSpec((1, 1, rows), lambda c, b, t: (b * nt + t, 0, 0)),
        ],
        out_specs=pl.BlockSpec((1, rows, HALF), lambda c, b, t: (c, b * nt + t, 0)),
        out_shape=jax.ShapeDtypeStruct((NC, B * N, HALF), jnp.float32),
    )(nodes, dis3)


EPT = NCHUNK * CH


def _sc_spmm_body(snd2_hbm, rcv2_hbm, ns_hbm, z_hbm,
                  p_hbm,
                  si2, ri2, rows0, rows1,
                  acc_sh, g0, g1, s0, s1):
    c = lax.axis_index("c")
    w = lax.axis_index("s")

    def batch(b, car):
        for j in range(RPT // CH):
            pltpu.sync_copy(z_hbm, acc_sh.at[pl.ds(w * RPT + j * CH, CH)])
        pltpu.sync_copy(snd2_hbm.at[b, pl.ds(w * NCHUNK, NCHUNK)], si2)
        pltpu.sync_copy(rcv2_hbm.at[b, pl.ds(w * NCHUNK, NCHUNK)], ri2)
        base = b * N + c * (B * N)

        def gi(k, car2):
            for j in range(CH // 16):
                sl = pl.ds(j * 16, 16)
                si2[k, sl] = si2[k, sl] + base
            return car2

        lax.fori_loop(0, NCHUNK, gi, 0)
        plsc.subcore_barrier()

        bufs = (rows0, rows1)
        gsem = (g0, g1)
        ssem = (s0, s1)
        gd = [pltpu.async_copy(ns_hbm.at[si2.at[k]], bufs[k], gsem[k])
              for k in range(2)]
        sd = [None, None]
        for k in range(NCHUNK):
            p = k & 1
            gd[p].wait()
            sd[p] = pltpu.async_copy(bufs[p], acc_sh.at[ri2.at[k]],
                                     ssem[p], add=True)
            if k + 2 < NCHUNK:
                sd[p].wait()
                gd[p] = pltpu.async_copy(ns_hbm.at[si2.at[k + 2]],
                                         bufs[p], gsem[p])
        sd[0].wait()
        sd[1].wait()
        plsc.subcore_barrier()
        sl = pl.ds(w * RPT, RPT)
        pltpu.sync_copy(acc_sh.at[sl], p_hbm.at[c, b, sl])
        plsc.subcore_barrier()
        return car

    lax.fori_loop(0, B, batch, 0)


_sc_spmm = functools.partial(
    pl.kernel,
    mesh=plsc.VectorSubcoreMesh(core_axis_name="c", subcore_axis_name="s"),
    compiler_params=pltpu.CompilerParams(use_tc_tiling_on_sc=False,
                                         needs_layout_passes=False),
    out_type=jax.ShapeDtypeStruct((NC, B, NROWS, HALF), jnp.float32),
    scratch_types=[
        pltpu.VMEM((NCHUNK, CH), jnp.int32),
        pltpu.VMEM((NCHUNK, CH), jnp.int32),
        pltpu.VMEM((CH, HALF), jnp.float32),
        pltpu.VMEM((CH, HALF), jnp.float32),
        pltpu.VMEM_SHARED((NROWS, HALF), jnp.float32),
        pltpu.SemaphoreType.DMA,
        pltpu.SemaphoreType.DMA,
        pltpu.SemaphoreType.DMA,
        pltpu.SemaphoreType.DMA,
    ],
)(_sc_spmm_body)


def _mm_body(p0_ref, p1_ref, dis_ref, sdr_ref, w0_ref, w1_ref, bc_ref,
             z_ref, s1_ref, s2_ref):
    nt = pl.program_id(1)
    z = jnp.dot(p0_ref[0], w0_ref[...], preferred_element_type=jnp.float32)
    z = z + jnp.dot(p1_ref[0], w1_ref[...], preferred_element_type=jnp.float32)
    d = dis_ref[0, 0]
    z = z * d[:, None] + (d * sdr_ref[0, 0])[:, None] * bc_ref[0, 0][None, :]
    z_ref[0] = z
    rows = nt * RT + lax.broadcasted_iota(jnp.int32, (RT, 1), 0)
    zm = jnp.where(rows < N, z, 0.0)
    s1 = jnp.sum(zm, axis=0)
    s2 = jnp.sum(zm * zm, axis=0)

    @pl.when(nt == 0)
    def _():
        s1_ref[0, 0] = s1
        s2_ref[0, 0] = s2

    @pl.when(nt > 0)
    def _():
        s1_ref[0, 0] += s1
        s2_ref[0, 0] += s2


def _mm_call(p0, p1, dis, sdr, w0, w1, bcat):
    nw = w0.shape[1]
    return pl.pallas_call(
        _mm_body,
        grid=(B, NT_D),
        in_specs=[
            pl.BlockSpec((1, RT, HALF), lambda b, t: (b, t, 0)),
            pl.BlockSpec((1, RT, HALF), lambda b, t: (b, t, 0)),
            pl.BlockSpec((1, 1, RT), lambda b, t: (b * NT_D + t, 0, 0)),
            pl.BlockSpec((1, 1, RT), lambda b, t: (b * NT_D + t, 0, 0)),
            pl.BlockSpec((HALF, nw), lambda b, t: (0, 0)),
            pl.BlockSpec((HALF, nw), lambda b, t: (0, 0)),
            pl.BlockSpec((1, 1, nw), lambda b, t: (0, 0, 0)),
        ],
        out_specs=[
            pl.BlockSpec((1, RT, nw), lambda b, t: (b, t, 0)),
            pl.BlockSpec((1, 1, nw), lambda b, t: (b, 0, 0)),
            pl.BlockSpec((1, 1, nw), lambda b, t: (b, 0, 0)),
        ],
        out_shape=[
            jax.ShapeDtypeStruct((B, NROWS, nw), jnp.float32),
            jax.ShapeDtypeStruct((B, 1, nw), jnp.float32),
            jax.ShapeDtypeStruct((B, 1, nw), jnp.float32),
        ],
    )(p0, p1, dis, sdr, w0, w1, bcat)


def _norm_body(z_ref, s1_ref, s2_ref, dis_ref, sds_ref, gc_ref, oc_ref,
               xp_ref, t_ref, sw_ref):
    nt = pl.program_id(1)
    inv = 1.0 / N
    mu = s1_ref[0, 0] * inv
    var = s2_ref[0, 0] * inv - mu * mu
    rstd = lax.rsqrt(var + EPS)
    x = (z_ref[0] - mu[None, :]) * rstd[None, :] * gc_ref[0, 0][None, :] \
        + oc_ref[0, 0][None, :]
    x = jnp.maximum(x, 0.0)
    wv = dis_ref[0, 0] * sds_ref[0, 0]
    tp = jnp.dot(wv[None, :], x, preferred_element_type=jnp.float32)
    xp_ref[0, 0] = x[:, FS]

    @pl.when(nt == 0)
    def _():
        t_ref[0, 0] = tp[0]
        sw_ref[0, 0] = jnp.full(tp[0].shape, jnp.sum(wv), jnp.float32)

    @pl.when(nt > 0)
    def _():
        t_ref[0, 0] += tp[0]
        sw_ref[0, 0] += jnp.full(tp[0].shape, jnp.sum(wv), jnp.float32)


def _norm_call(z, s1, s2, dis, sds, gcat, ocat):
    nw = z.shape[2]
    return pl.pallas_call(
        _norm_body,
        grid=(B, NT_D),
        in_specs=[
            pl.BlockSpec((1, RT, nw), lambda b, t: (b, t, 0)),
            pl.BlockSpec((1, 1, nw), lambda b, t: (b, 0, 0)),
            pl.BlockSpec((1, 1, nw), lambda b, t: (b, 0, 0)),
            pl.BlockSpec((1, 1, RT), lambda b, t: (b * NT_D + t, 0, 0)),
            pl.BlockSpec((1, 1, RT), lambda b, t: (b * NT_D + t, 0, 0)),
            pl.BlockSpec((1, 1, nw), lambda b, t: (0, 0, 0)),
            pl.BlockSpec((1, 1, nw), lambda b, t: (0, 0, 0)),
        ],
        out_specs=[
            pl.BlockSpec((1, 1, RT), lambda b, t: (b * NT_D + t, 0, 0)),
            pl.BlockSpec((1, 1, nw), lambda b, t: (b, 0, 0)),
            pl.BlockSpec((1, 1, nw), lambda b, t: (b, 0, 0)),
        ],
        out_shape=[
            jax.ShapeDtypeStruct((B * NT_D, 1, RT), jnp.float32),
            jax.ShapeDtypeStruct((B, 1, nw), jnp.float32),
            jax.ShapeDtypeStruct((B, 1, nw), jnp.float32),
        ],
    )(z, s1, s2, dis, sds, gcat, ocat)


def _sc_pol_body(snd2_hbm, rcv2_hbm, dis_hbm, xp_hbm, sdr_hbm, wb_hbm,
                 lg_hbm,
                 si2, ri2, msg0, msg1, disv, xpv, accv, sdrv, outv, zer_v,
                 wbv, acc_sh, sem0, sem1):
    c = lax.axis_index("c")
    w = lax.axis_index("s")
    _fill(zer_v, RPT, 0.0, jnp.float32)
    pltpu.sync_copy(wb_hbm, wbv)

    def batch(i, car):
        b = 2 * c + i
        sl = pl.ds(w * RPT, RPT)
        pltpu.sync_copy(zer_v, acc_sh.at[sl])
        pltpu.sync_copy(dis_hbm.at[b], disv)
        pltpu.sync_copy(xp_hbm.at[b], xpv)
        pltpu.sync_copy(sdr_hbm.at[b, sl], sdrv)
        pltpu.sync_copy(snd2_hbm.at[b, pl.ds(w * NCHUNK, NCHUNK)], si2)
        pltpu.sync_copy(rcv2_hbm.at[b, pl.ds(w * NCHUNK, NCHUNK)], ri2)
        plsc.subcore_barrier()
        msgs = (msg0, msg1)
        sems = (sem0, sem1)
        sd = [None, None]
        for k in range(NCHUNK):
            p = k & 1
            if sd[p] is not None:
                sd[p].wait()
            for j in range(CH // 16):
                s16 = pl.ds(j * 16, 16)
                sv = si2[k, s16]
                msgs[p][s16] = (plsc.load_gather(disv, [sv])
                                * plsc.load_gather(disv, [ri2[k, s16]])
                                * plsc.load_gather(xpv, [sv]))
            sd[p] = pltpu.async_copy(msgs[p], acc_sh.at[ri2.at[k]],
                                     sems[p], add=True)
        sd[0].wait()
        sd[1].wait()
        plsc.subcore_barrier()
        pltpu.sync_copy(acc_sh.at[sl], accv)
        wvec = wbv[...]
        w2 = wvec[0]
        b2 = wvec[1]

        def vv(j, car2):
            s16 = pl.ds(j * 16, 16)
            dsl = disv[pl.ds(w * RPT + j * 16, 16)]
            outv[s16] = accv[s16] * w2 + dsl * sdrv[s16] * b2
            return car2

        lax.fori_loop(0, RPT // 16, vv, 0)
        pltpu.sync_copy(outv, lg_hbm.at[b, sl])
        plsc.subcore_barrier()
        return car

    lax.fori_loop(0, 2, batch, 0)


_sc_pol = functools.partial(
    pl.kernel,
    mesh=plsc.VectorSubcoreMesh(core_axis_name="c", subcore_axis_name="s"),
    compiler_params=pltpu.CompilerParams(use_tc_tiling_on_sc=False,
                                         needs_layout_passes=False),
    out_type=jax.ShapeDtypeStruct((B, NROWS), jnp.float32),
    scratch_types=[
        pltpu.VMEM((NCHUNK, CH), jnp.int32),
        pltpu.VMEM((NCHUNK, CH), jnp.int32),
        pltpu.VMEM((CH,), jnp.float32),
        pltpu.VMEM((CH,), jnp.float32),
        pltpu.VMEM((NROWS,), jnp.float32),
        pltpu.VMEM((NROWS,), jnp.float32),
        pltpu.VMEM((RPT,), jnp.float32),
        pltpu.VMEM((RPT,), jnp.float32),
        pltpu.VMEM((RPT,), jnp.float32),
        pltpu.VMEM((RPT,), jnp.float32),
        pltpu.VMEM((16,), jnp.float32),
        pltpu.VMEM_SHARED((NROWS,), jnp.float32),
        pltpu.SemaphoreType.DMA,
        pltpu.SemaphoreType.DMA,
    ],
)(_sc_pol_body)


def _fin_body(t_ref, w2_ref, sw_ref, bv_ref, v_ref):
    v_ref[...] = jnp.dot(t_ref[...], w2_ref[...],
                         preferred_element_type=jnp.float32) \
        + sw_ref[:, 0:1] * bv_ref[...]


def _fin_call(t, w2p, sw, bv2):
    return pl.pallas_call(
        _fin_body,
        out_shape=jax.ShapeDtypeStruct((B, FS), jnp.float32),
    )(t, w2p, sw, bv2)


def kernel(s, W_pi1, b_pi1, g_pi, o_pi, W_pi2, b_pi2, W_v1, b_v1, g_v, o_v,
           W_v2, b_v2):
    f32 = jnp.float32
    nodes = s[:, :SLEFT].reshape(B, N, EMB)
    snd = s[:, SLEFT:SLEFT + NE].astype(jnp.int32)
    rcv = s[:, SLEFT + NE:SLEFT + 2 * NE].astype(jnp.int32)
    pad = EP - NE
    snd_p = jnp.pad(snd, ((0, 0), (0, pad)))
    rcv_pad = (N + (jnp.arange(pad, dtype=jnp.int32) % 16))[None, :]
    rcv_p = jnp.concatenate([rcv, jnp.broadcast_to(rcv_pad, (B, pad))], axis=1)

    snd3 = snd_p.reshape(B, NS * NCHUNK, CH)
    rcv3 = rcv_p.reshape(B, NS * NCHUNK, CH)
    dis, sdr, sds = _sc_pre(snd3, rcv3)
    ns = _prep_call(nodes, dis[:, :N].reshape(B * 10, 1, N // 10)).reshape(
        NC * B * N, HALF)
    z128 = jnp.zeros((CH, HALF), f32)
    p = _sc_spmm(snd3, rcv3, ns, z128)

    nw = 640
    wcat = jnp.concatenate([W_v1, W_pi1], axis=1)
    wcat = jnp.pad(wcat, ((0, 0), (0, nw - FS - 1)))
    bcat = jnp.pad(jnp.concatenate([b_v1, b_pi1]), (0, nw - FS - 1))[None, None, :]
    gcat = jnp.pad(jnp.concatenate([g_v, g_pi]), (0, nw - FS - 1))[None, None, :]
    ocat = jnp.pad(jnp.concatenate([o_v, o_pi]), (0, nw - FS - 1))[None, None, :]

    dis_rt = dis.reshape(B * NT_D, 1, RT)
    sdr_rt = sdr.reshape(B * NT_D, 1, RT)
    sds_rt = sds.reshape(B * NT_D, 1, RT)
    z, s1, s2 = _mm_call(p[0], p[1], dis_rt, sdr_rt,
                         wcat[:HALF], wcat[HALF:], bcat)
    xp3, t3, sw3 = _norm_call(z, s1, s2, dis_rt, sds_rt, gcat, ocat)
    xp = xp3.reshape(B, NROWS)
    t = t3.reshape(B, nw)
    sw = sw3.reshape(B, nw)

    wb = jnp.zeros((16,), f32).at[0].set(W_pi2[0, 0]).at[1].set(b_pi2[0])
    lg = _sc_pol(snd3, rcv3, dis, xp, sdr, wb)
    logits = lg[:, :N - 2]

    w2p = jnp.pad(W_v2, ((0, nw - FS), (0, 0)))
    v = _fin_call(t, w2p, sw, b_v2[None, :])
    return (v, logits)
